# baseline (speedup 1.0000x reference)
import jax
import jax.numpy as jnp
from jax import lax
from jax.experimental import pallas as pl
from jax.experimental.pallas import tpu as pltpu
from jax.experimental.pallas import tpu_sc as plsc

N = 100000
DA = 128
L = 16
NC = 2
NS = 16
NW = NC * NS
R = 160
NFULL = N // R
K = DA // L


def _body(x_hbm, v_hbm, u_hbm, b_hbm, beff_hbm, eff_hbm, y_hbm,
          xin0, xin1, yout0, yout1, effb0, effb1,
          vvm, uvm, bvm, beffvm, isem0, isem1, osem0, osem1):
    wid = lax.axis_index("s") * NC + lax.axis_index("c")

    pltpu.sync_copy(v_hbm, vvm)
    pltpu.sync_copy(u_hbm, uvm)
    pltpu.sync_copy(b_hbm, bvm)
    pltpu.sync_copy(beff_hbm, beffvm)

    vv = [vvm[pl.ds(k * L, L)] for k in range(K)]
    uu = [uvm[pl.ds(k * L, L)] for k in range(K)]
    bb = [bvm[pl.ds(k * L, L)] for k in range(K)]
    beffv = beffvm[...]
    lane = lax.iota(jnp.int32, L)

    def do_rows(xin, yout, effb, nrows):
        @plsc.parallel_loop(0, nrows // L, unroll=2)
        def group_body(g):
            base_r = g * L
            z16 = jnp.zeros((L,), jnp.float32)
            for i in range(L):
                r = base_r + i
                ps = [xin[r, pl.ds(k * L, L)] * vv[k] for k in range(K)]
                while len(ps) > 1:
                    ps = [ps[j] + ps[j + 1] for j in range(0, len(ps), 2)]
                zs = jnp.sum(ps[0])
                z16 = jnp.where(lane == i, zs, z16)
            eff16 = 1.0 / (1.0 + jnp.exp(-(z16 + beffv)))
            effb[pl.ds(base_r, L)] = eff16
            for i in range(L):
                r = base_r + i
                e = eff16[i]
                for k in range(K):
                    yout[r, pl.ds(k * L, L)] = e * (
                        uu[k] * xin[r, pl.ds(k * L, L)] + bb[k])

    nchunks = (NFULL - 1 - wid) // NW + 1

    bufs = ((xin0, yout0, effb0, isem0, osem0),
            (xin1, yout1, effb1, isem1, osem1))

    pltpu.make_async_copy(x_hbm.at[pl.ds(wid * R, R)], xin0, isem0).start()

    def chunk_body(t, carry):
        base = (wid + t * NW) * R

        def phase(cur, nxt):
            xin, yout, effb, isem, osem = cur
            nxin, _, _, nisem, _ = nxt
            pltpu.make_async_copy(x_hbm.at[pl.ds(base, R)], xin, isem).wait()

            @pl.when(t + 1 < nchunks)
            def _prefetch():
                nbase = (wid + (t + 1) * NW) * R
                pltpu.make_async_copy(
                    x_hbm.at[pl.ds(nbase, R)], nxin, nisem).start()

            @pl.when(t >= 2)
            def _drain():
                pltpu.make_async_copy(
                    yout, y_hbm.at[pl.ds(base, R)], osem).wait()
                pltpu.make_async_copy(
                    effb, eff_hbm.at[pl.ds(base, R)], osem).wait()

            do_rows(xin, yout, effb, R)
            pltpu.make_async_copy(yout, y_hbm.at[pl.ds(base, R)], osem).start()
            pltpu.make_async_copy(effb, eff_hbm.at[pl.ds(base, R)], osem).start()

        @pl.when(t % 2 == 0)
        def _even():
            phase(bufs[0], bufs[1])

        @pl.when(t % 2 == 1)
        def _odd():
            phase(bufs[1], bufs[0])

        return carry

    lax.fori_loop(0, nchunks, chunk_body, 0)

    for (_, yout, effb, _, osem) in bufs:
        pltpu.make_async_copy(yout, y_hbm.at[pl.ds(0, R)], osem).wait()
        pltpu.make_async_copy(effb, eff_hbm.at[pl.ds(0, R)], osem).wait()

@jax.jit
def _run(x, v, u, b, beff16):
    mesh = plsc.VectorSubcoreMesh(core_axis_name="c", subcore_axis_name="s",
                                  num_cores=NC, num_subcores=NS)
    eff, y = pl.kernel(
        _body,
        out_type=(jax.ShapeDtypeStruct((N,), jnp.float32),
                  jax.ShapeDtypeStruct((N, DA), jnp.float32)),
        mesh=mesh,
        compiler_params=pltpu.CompilerParams(needs_layout_passes=False),
        scratch_types=(
            pltpu.VMEM((R, DA), jnp.float32),
            pltpu.VMEM((R, DA), jnp.float32),
            pltpu.VMEM((R, DA), jnp.float32),
            pltpu.VMEM((R, DA), jnp.float32),
            pltpu.VMEM((R,), jnp.float32),
            pltpu.VMEM((R,), jnp.float32),
            pltpu.VMEM((DA,), jnp.float32),
            pltpu.VMEM((DA,), jnp.float32),
            pltpu.VMEM((DA,), jnp.float32),
            pltpu.VMEM((L,), jnp.float32),
            pltpu.SemaphoreType.DMA,
            pltpu.SemaphoreType.DMA,
            pltpu.SemaphoreType.DMA,
            pltpu.SemaphoreType.DMA,
        ),
    )(x, v, u, b, beff16)
    return eff, y


def kernel(x, n, W_eff, b_eff, w_t, b_t):
    v = n * W_eff[0]
    u = w_t * n
    beff16 = jnp.broadcast_to(b_eff[0], (L,))
    eff, y = _run(x, v, u, b_t, beff16)
    return (eff.reshape(N, 1), y)


# speedup vs baseline: 1.7127x; 1.7127x over previous
import jax
import jax.numpy as jnp
from jax import lax
from jax.experimental import pallas as pl
from jax.experimental.pallas import tpu as pltpu
from jax.experimental.pallas import tpu_sc as plsc

N = 100000
DA = 128
L = 16
NC = 2
NS = 16
NW = NC * NS
R = 160
NFULL = N // R
K = DA // L


def _body(x_hbm, v_hbm, u_hbm, b_hbm, beff_hbm, eff_hbm, y_hbm,
          xin0, xin1, yout0, yout1, effb0, effb1,
          vvm, uvm, bvm, beffvm, isem0, isem1, osem0, osem1):
    wid = lax.axis_index("s") * NC + lax.axis_index("c")

    pltpu.sync_copy(v_hbm, vvm)
    pltpu.sync_copy(u_hbm, uvm)
    pltpu.sync_copy(b_hbm, bvm)
    pltpu.sync_copy(beff_hbm, beffvm)

    vv = [vvm[pl.ds(k * L, L)] for k in range(K)]
    uu = [uvm[pl.ds(k * L, L)] for k in range(K)]
    bb = [bvm[pl.ds(k * L, L)] for k in range(K)]
    beffv = beffvm[...]
    lane = lax.iota(jnp.int32, L)

    def do_rows(xin, yout, effb, nrows):
        @plsc.parallel_loop(0, nrows // L)
        def group_body(g):
            base_r = g * L
            z16 = jnp.zeros((L,), jnp.float32)
            for i in range(L):
                r = base_r + i
                ps = [xin[r, pl.ds(k * L, L)] * vv[k] for k in range(K)]
                while len(ps) > 1:
                    ps = [ps[j] + ps[j + 1] for j in range(0, len(ps), 2)]
                zs = jnp.sum(ps[0])
                z16 = jnp.where(lane == i, zs, z16)
            eff16 = 1.0 / (1.0 + jnp.exp(-(z16 + beffv)))
            effb[pl.ds(base_r, L)] = eff16
            for i in reversed(range(L)):
                r = base_r + i
                e = eff16[i]
                for k in range(K):
                    yout[r, pl.ds(k * L, L)] = e * (
                        uu[k] * xin[r, pl.ds(k * L, L)] + bb[k])

    nchunks = (NFULL - 1 - wid) // NW + 1

    bufs = ((xin0, yout0, effb0, isem0, osem0),
            (xin1, yout1, effb1, isem1, osem1))

    pltpu.make_async_copy(x_hbm.at[pl.ds(wid * R, R)], xin0, isem0).start()

    def chunk_body(t, carry):
        base = (wid + t * NW) * R

        def phase(cur, nxt):
            xin, yout, effb, isem, osem = cur
            nxin, _, _, nisem, _ = nxt
            pltpu.make_async_copy(x_hbm.at[pl.ds(base, R)], xin, isem).wait()

            @pl.when(t + 1 < nchunks)
            def _prefetch():
                nbase = (wid + (t + 1) * NW) * R
                pltpu.make_async_copy(
                    x_hbm.at[pl.ds(nbase, R)], nxin, nisem).start()

            @pl.when(t >= 2)
            def _drain():
                pltpu.make_async_copy(
                    yout, y_hbm.at[pl.ds(base, R)], osem).wait()
                pltpu.make_async_copy(
                    effb, eff_hbm.at[pl.ds(base, R)], osem).wait()

            do_rows(xin, yout, effb, R)
            pltpu.make_async_copy(yout, y_hbm.at[pl.ds(base, R)], osem).start()
            pltpu.make_async_copy(effb, eff_hbm.at[pl.ds(base, R)], osem).start()

        @pl.when(t % 2 == 0)
        def _even():
            phase(bufs[0], bufs[1])

        @pl.when(t % 2 == 1)
        def _odd():
            phase(bufs[1], bufs[0])

        return carry

    lax.fori_loop(0, nchunks, chunk_body, 0)

    for (_, yout, effb, _, osem) in bufs:
        pltpu.make_async_copy(yout, y_hbm.at[pl.ds(0, R)], osem).wait()
        pltpu.make_async_copy(effb, eff_hbm.at[pl.ds(0, R)], osem).wait()

@jax.jit
def _run(x, v, u, b, beff16):
    mesh = plsc.VectorSubcoreMesh(core_axis_name="c", subcore_axis_name="s",
                                  num_cores=NC, num_subcores=NS)
    eff, y = pl.kernel(
        _body,
        out_type=(jax.ShapeDtypeStruct((N,), jnp.float32),
                  jax.ShapeDtypeStruct((N, DA), jnp.float32)),
        mesh=mesh,
        compiler_params=pltpu.CompilerParams(needs_layout_passes=False),
        scratch_types=(
            pltpu.VMEM((R, DA), jnp.float32),
            pltpu.VMEM((R, DA), jnp.float32),
            pltpu.VMEM((R, DA), jnp.float32),
            pltpu.VMEM((R, DA), jnp.float32),
            pltpu.VMEM((R,), jnp.float32),
            pltpu.VMEM((R,), jnp.float32),
            pltpu.VMEM((DA,), jnp.float32),
            pltpu.VMEM((DA,), jnp.float32),
            pltpu.VMEM((DA,), jnp.float32),
            pltpu.VMEM((L,), jnp.float32),
            pltpu.SemaphoreType.DMA,
            pltpu.SemaphoreType.DMA,
            pltpu.SemaphoreType.DMA,
            pltpu.SemaphoreType.DMA,
        ),
    )(x, v, u, b, beff16)
    return eff, y


def kernel(x, n, W_eff, b_eff, w_t, b_t):
    v = n * W_eff[0]
    u = w_t * n
    beff16 = jnp.broadcast_to(b_eff[0], (L,))
    eff, y = _run(x, v, u, b_t, beff16)
    return (eff.reshape(N, 1), y)


# speedup vs baseline: 1.7221x; 1.0055x over previous
import jax
import jax.numpy as jnp
from jax import lax
from jax.experimental import pallas as pl
from jax.experimental.pallas import tpu as pltpu
from jax.experimental.pallas import tpu_sc as plsc

N = 100000
DA = 128
L = 16
NC = 2
NS = 16
NW = NC * NS
R = 160
NFULL = N // R
K = DA // L


def _body(x_hbm, v_hbm, u_hbm, b_hbm, beff_hbm, eff_hbm, y_hbm,
          xin0, xin1, yout0, yout1, effb0, effb1,
          vvm, uvm, bvm, beffvm, isem0, isem1, osem0, osem1):
    wid = lax.axis_index("s") * NC + lax.axis_index("c")

    pltpu.sync_copy(v_hbm, vvm)
    pltpu.sync_copy(u_hbm, uvm)
    pltpu.sync_copy(b_hbm, bvm)
    pltpu.sync_copy(beff_hbm, beffvm)

    vv = [vvm[pl.ds(k * L, L)] for k in range(K)]
    uu = [uvm[pl.ds(k * L, L)] for k in range(K)]
    bb = [bvm[pl.ds(k * L, L)] for k in range(K)]
    beffv = beffvm[...]
    lane = lax.iota(jnp.int32, L)

    def do_rows(xin, yout, effb, nrows):
        @plsc.parallel_loop(0, nrows // L)
        def group_body(g):
            base_r = g * L
            z16 = jnp.zeros((L,), jnp.float32)
            for i in range(L):
                r = base_r + i
                ps = [xin[r, pl.ds(k * L, L)] * vv[k] for k in range(K)]
                while len(ps) > 1:
                    ps = [ps[j] + ps[j + 1] for j in range(0, len(ps), 2)]
                zs = jnp.sum(ps[0])
                z16 = jnp.where(lane == i, zs, z16)
            eff16 = 1.0 / (1.0 + jnp.exp(-(z16 + beffv)))
            effb[pl.ds(base_r, L)] = eff16
            for i in range(L):
                r = base_r + i
                e = eff16[i]
                for k in range(K):
                    yout[r, pl.ds(k * L, L)] = e * (
                        uu[k] * xin[r, pl.ds(k * L, L)] + bb[k])

    nchunks = (NFULL - 1 - wid) // NW + 1

    bufs = ((xin0, yout0, effb0, isem0, osem0),
            (xin1, yout1, effb1, isem1, osem1))

    pltpu.make_async_copy(x_hbm.at[pl.ds(wid * R, R)], xin0, isem0).start()

    def chunk_body(t, carry):
        base = (wid + t * NW) * R

        def phase(cur, nxt):
            xin, yout, effb, isem, osem = cur
            nxin, _, _, nisem, _ = nxt
            pltpu.make_async_copy(x_hbm.at[pl.ds(base, R)], xin, isem).wait()

            @pl.when(t + 1 < nchunks)
            def _prefetch():
                nbase = (wid + (t + 1) * NW) * R
                pltpu.make_async_copy(
                    x_hbm.at[pl.ds(nbase, R)], nxin, nisem).start()

            @pl.when(t >= 2)
            def _drain():
                pltpu.make_async_copy(
                    yout, y_hbm.at[pl.ds(base, R)], osem).wait()
                pltpu.make_async_copy(
                    effb, eff_hbm.at[pl.ds(base, R)], osem).wait()

            do_rows(xin, yout, effb, R)
            pltpu.make_async_copy(yout, y_hbm.at[pl.ds(base, R)], osem).start()
            pltpu.make_async_copy(effb, eff_hbm.at[pl.ds(base, R)], osem).start()

        @pl.when(t % 2 == 0)
        def _even():
            phase(bufs[0], bufs[1])

        @pl.when(t % 2 == 1)
        def _odd():
            phase(bufs[1], bufs[0])

        return carry

    lax.fori_loop(0, nchunks, chunk_body, 0)

    for (_, yout, effb, _, osem) in bufs:
        pltpu.make_async_copy(yout, y_hbm.at[pl.ds(0, R)], osem).wait()
        pltpu.make_async_copy(effb, eff_hbm.at[pl.ds(0, R)], osem).wait()

@jax.jit
def _run(x, v, u, b, beff16):
    mesh = plsc.VectorSubcoreMesh(core_axis_name="c", subcore_axis_name="s",
                                  num_cores=NC, num_subcores=NS)
    eff, y = pl.kernel(
        _body,
        out_type=(jax.ShapeDtypeStruct((N,), jnp.float32),
                  jax.ShapeDtypeStruct((N, DA), jnp.float32)),
        mesh=mesh,
        compiler_params=pltpu.CompilerParams(needs_layout_passes=False),
        scratch_types=(
            pltpu.VMEM((R, DA), jnp.float32),
            pltpu.VMEM((R, DA), jnp.float32),
            pltpu.VMEM((R, DA), jnp.float32),
            pltpu.VMEM((R, DA), jnp.float32),
            pltpu.VMEM((R,), jnp.float32),
            pltpu.VMEM((R,), jnp.float32),
            pltpu.VMEM((DA,), jnp.float32),
            pltpu.VMEM((DA,), jnp.float32),
            pltpu.VMEM((DA,), jnp.float32),
            pltpu.VMEM((L,), jnp.float32),
            pltpu.SemaphoreType.DMA,
            pltpu.SemaphoreType.DMA,
            pltpu.SemaphoreType.DMA,
            pltpu.SemaphoreType.DMA,
        ),
    )(x, v, u, b, beff16)
    return eff, y


def kernel(x, n, W_eff, b_eff, w_t, b_t):
    v = n * W_eff[0]
    u = w_t * n
    beff16 = jnp.broadcast_to(b_eff[0], (L,))
    eff, y = _run(x, v, u, b_t, beff16)
    return (eff.reshape(N, 1), y)
